# Initial kernel scaffold; baseline (speedup 1.0000x reference)
import functools

import jax
import jax.numpy as jnp
from jax import lax
from jax.experimental import pallas as pl
from jax.experimental.pallas import tpu as pltpu
from jax.experimental.pallas import tpu_sc as plsc

N = 10000
F = 128
E = 320000
NC = 2
NS = 16
NW = NC * NS
EPW = E // NW
B = 125
NB = EPW // B
RPT = N // NS
RB = 1000

_mesh = plsc.VectorSubcoreMesh(core_axis_name="c", subcore_axis_name="s")


@functools.partial(
    pl.kernel,
    mesh=_mesh,
    out_type=jax.ShapeDtypeStruct((NW, N), jnp.float32),
    scratch_types=[
        pltpu.VMEM((EPW,), jnp.int32),
        pltpu.VMEM((N,), jnp.float32),
    ],
)
def _sc_degree(dst_hbm, out_hbm, dst_v, acc_v):
    wid = lax.axis_index("s") * NC + lax.axis_index("c")
    pltpu.sync_copy(dst_hbm.at[pl.ds(wid * EPW, EPW)], dst_v)
    zero = jnp.zeros((16,), jnp.float32)
    one = jnp.ones((16,), jnp.float32)

    def zbody(i, c):
        acc_v[pl.ds(i * 16, 16)] = zero
        return c

    lax.fori_loop(0, N // 16, zbody, 0)

    def body(i, c):
        ids = dst_v[pl.ds(i * 16, 16)]
        plsc.addupdate_scatter(acc_v, [ids], one)
        return c

    lax.fori_loop(0, EPW // 16, body, 0)
    pltpu.sync_copy(acc_v, out_hbm.at[wid])


@functools.partial(
    pl.kernel,
    mesh=_mesh,
    out_type=jax.ShapeDtypeStruct((NC * N, F), jnp.float32),
    scratch_types=[
        pltpu.VMEM((NB, B), jnp.int32),
        pltpu.VMEM((NB, B), jnp.int32),
        pltpu.VMEM((B, F), jnp.float32),
        pltpu.VMEM((B, F), jnp.float32),
        pltpu.VMEM_SHARED((N, F), jnp.float32),
        pltpu.SemaphoreType.DMA,
        pltpu.SemaphoreType.DMA,
    ],
)
def _sc_agg_rows(y_hbm, src_hbm, dst_hbm, zeros_hbm, out_hbm,
                 src_v, dst_v, rb0, rb1, acc_sh, sem0, sem1):
    cid = lax.axis_index("c")
    sid = lax.axis_index("s")
    wid = sid * NC + cid
    pltpu.sync_copy(src_hbm.at[pl.ds(wid * NB, NB)], src_v)
    pltpu.sync_copy(dst_hbm.at[pl.ds(wid * NB, NB)], dst_v)
    pltpu.sync_copy(zeros_hbm, acc_sh.at[pl.ds(sid * RPT, RPT)])
    plsc.subcore_barrier()

    def step(j, rb, sem):
        pltpu.async_copy(y_hbm.at[src_v.at[j]], rb, sem).wait()
        pltpu.sync_copy(rb, acc_sh.at[dst_v.at[j]], add=True)

    def body(j, c):
        step(2 * j, rb0, sem0)
        step(2 * j + 1, rb1, sem1)
        return c

    lax.fori_loop(0, NB // 2, body, 0)
    plsc.subcore_barrier()
    pltpu.sync_copy(acc_sh.at[pl.ds(sid * RPT, RPT)],
                    out_hbm.at[pl.ds(cid * N + sid * RPT, RPT)])


@functools.partial(
    pl.kernel,
    mesh=_mesh,
    out_type=jax.ShapeDtypeStruct((NW, N), jnp.float32),
    scratch_types=[
        pltpu.VMEM((N,), jnp.float32),
        pltpu.VMEM((EPW,), jnp.int32),
        pltpu.VMEM((EPW,), jnp.int32),
        pltpu.VMEM((N,), jnp.float32),
    ],
)
def _sc_agg_scalar(q_hbm, src_hbm, dst_hbm, out_hbm, q_v, src_v, dst_v, acc_v):
    wid = lax.axis_index("s") * NC + lax.axis_index("c")
    pltpu.sync_copy(q_hbm, q_v)
    pltpu.sync_copy(src_hbm.at[pl.ds(wid * EPW, EPW)], src_v)
    pltpu.sync_copy(dst_hbm.at[pl.ds(wid * EPW, EPW)], dst_v)
    zero = jnp.zeros((16,), jnp.float32)

    def zbody(i, c):
        acc_v[pl.ds(i * 16, 16)] = zero
        return c

    lax.fori_loop(0, N // 16, zbody, 0)

    def body(i, c):
        s_ids = src_v[pl.ds(i * 16, 16)]
        d_ids = dst_v[pl.ds(i * 16, 16)]
        vals = plsc.load_gather(q_v, [s_ids])
        plsc.addupdate_scatter(acc_v, [d_ids], vals)
        return c

    lax.fori_loop(0, EPW // 16, body, 0)
    pltpu.sync_copy(acc_v, out_hbm.at[wid])


def _tc1_body(x_ref, w_ref, deg_ref, y_ref, dis_ref):
    deg = 1.0 + jnp.sum(deg_ref[...], axis=1, keepdims=True)
    dis = lax.rsqrt(deg)
    xw = jnp.dot(x_ref[...], w_ref[...], preferred_element_type=jnp.float32)
    y_ref[...] = xw * dis
    dis_ref[...] = dis


def _tc2_body(agg_ref, y_ref, dis_ref, b1_ref, w2_ref, q_ref):
    agg = agg_ref[0] + agg_ref[1]
    h = jnp.maximum(dis_ref[...] * (agg + y_ref[...]) + b1_ref[...], 0.0)
    q_ref[...] = jnp.dot(h, w2_ref[...],
                         preferred_element_type=jnp.float32) * dis_ref[...]


def _tc3_body(a_ref, q_ref, dis_ref, b2_ref, o_ref):
    s = jnp.sum(a_ref[...], axis=1, keepdims=True)
    o_ref[...] = dis_ref[...] * (s + q_ref[...]) + b2_ref[...]


def kernel(x, edge_index, W1, b1, W2, b2):
    ei = edge_index.astype(jnp.int32)
    src_flat = ei[0]
    dst_flat = ei[1]
    src2d = src_flat.reshape(NW * NB, B)
    dst2d = dst_flat.reshape(NW * NB, B)

    deg_parts = _sc_degree(dst_flat)

    grid = (N // RB,)
    y, dis = pl.pallas_call(
        _tc1_body,
        grid=grid,
        in_specs=[
            pl.BlockSpec((RB, F), lambda i: (i, 0)),
            pl.BlockSpec((F, F), lambda i: (0, 0)),
            pl.BlockSpec((RB, NW), lambda i: (i, 0)),
        ],
        out_specs=[
            pl.BlockSpec((RB, F), lambda i: (i, 0)),
            pl.BlockSpec((RB, 1), lambda i: (i, 0)),
        ],
        out_shape=[
            jax.ShapeDtypeStruct((N, F), jnp.float32),
            jax.ShapeDtypeStruct((N, 1), jnp.float32),
        ],
    )(x, W1, deg_parts.T)

    zeros_tile = jnp.zeros((RPT, F), jnp.float32)
    agg1 = _sc_agg_rows(y, src2d, dst2d, zeros_tile)

    q = pl.pallas_call(
        _tc2_body,
        grid=grid,
        in_specs=[
            pl.BlockSpec((NC, RB, F), lambda i: (0, i, 0)),
            pl.BlockSpec((RB, F), lambda i: (i, 0)),
            pl.BlockSpec((RB, 1), lambda i: (i, 0)),
            pl.BlockSpec((1, F), lambda i: (0, 0)),
            pl.BlockSpec((F, 1), lambda i: (0, 0)),
        ],
        out_specs=pl.BlockSpec((RB, 1), lambda i: (i, 0)),
        out_shape=jax.ShapeDtypeStruct((N, 1), jnp.float32),
    )(agg1.reshape(NC, N, F), y, dis, b1.reshape(1, F), W2)

    agg2_parts = _sc_agg_scalar(q.reshape(N), src_flat, dst_flat)

    out = pl.pallas_call(
        _tc3_body,
        grid=grid,
        in_specs=[
            pl.BlockSpec((RB, NW), lambda i: (i, 0)),
            pl.BlockSpec((RB, 1), lambda i: (i, 0)),
            pl.BlockSpec((RB, 1), lambda i: (i, 0)),
            pl.BlockSpec((1, 1), lambda i: (0, 0)),
        ],
        out_specs=pl.BlockSpec((RB, 1), lambda i: (i, 0)),
        out_shape=jax.ShapeDtypeStruct((N, 1), jnp.float32),
    )(agg2_parts.T, q, dis, b2.reshape(1, 1))

    return out.reshape(N)


# speedup vs baseline: 35.3661x; 35.3661x over previous
import functools

import jax
import jax.numpy as jnp
from jax import lax
from jax.experimental import pallas as pl
from jax.experimental.pallas import tpu as pltpu
from jax.experimental.pallas import tpu_sc as plsc

N = 10000
F = 128
E = 320000
NC = 2
NS = 16
NW = NC * NS
EPW = E // NW
B = 125
NB = EPW // B
RPT = N // NS
RB = 1000

_mesh = plsc.VectorSubcoreMesh(core_axis_name="c", subcore_axis_name="s")
_sc_params = pltpu.CompilerParams(needs_layout_passes=False,
                                  use_tc_tiling_on_sc=False)


@functools.partial(
    pl.kernel,
    mesh=_mesh,
    compiler_params=_sc_params,
    out_type=jax.ShapeDtypeStruct((NW, N), jnp.float32),
    scratch_types=[
        pltpu.VMEM((EPW,), jnp.int32),
        pltpu.VMEM((N,), jnp.float32),
    ],
)
def _sc_degree(dst_hbm, out_hbm, dst_v, acc_v):
    wid = lax.axis_index("s") * NC + lax.axis_index("c")
    pltpu.sync_copy(dst_hbm.at[pl.ds(wid * EPW, EPW)], dst_v)
    zero = jnp.zeros((16,), jnp.float32)
    one = jnp.ones((16,), jnp.float32)

    def zbody(i, c):
        acc_v[pl.ds(i * 16, 16)] = zero
        return c

    lax.fori_loop(0, N // 16, zbody, 0)

    def body(i, c):
        ids = dst_v[pl.ds(i * 16, 16)]
        plsc.addupdate_scatter(acc_v, [ids], one)
        return c

    lax.fori_loop(0, EPW // 16, body, 0)
    pltpu.sync_copy(acc_v, out_hbm.at[wid])


@functools.partial(
    pl.kernel,
    mesh=_mesh,
    compiler_params=_sc_params,
    out_type=jax.ShapeDtypeStruct((NC * N, F), jnp.float32),
    scratch_types=[
        pltpu.VMEM((16, B), jnp.int32),
        pltpu.VMEM((16, B), jnp.int32),
        pltpu.VMEM((B, F), jnp.float32),
        pltpu.VMEM((B, F), jnp.float32),
        pltpu.VMEM_SHARED((N, F), jnp.float32),
        pltpu.SemaphoreType.DMA,
        pltpu.SemaphoreType.DMA,
    ],
)
def _sc_agg_rows(y_hbm, src_hbm, dst_hbm, zeros_hbm, out_hbm,
                 src_v, dst_v, rb0, rb1, acc_sh, sem0, sem1):
    cid = lax.axis_index("c")
    sid = lax.axis_index("s")
    wid = sid * NC + cid
    pltpu.sync_copy(zeros_hbm, acc_sh.at[pl.ds(sid * RPT, RPT)])
    plsc.subcore_barrier()

    def step(j, rb, sem):
        pltpu.async_copy(y_hbm.at[src_v.at[j]], rb, sem).wait()
        pltpu.sync_copy(rb, acc_sh.at[dst_v.at[j]], add=True)

    def chunk(k, c):
        pltpu.sync_copy(src_hbm.at[pl.ds(wid * NB + k * 16, 16)], src_v)
        pltpu.sync_copy(dst_hbm.at[pl.ds(wid * NB + k * 16, 16)], dst_v)

        def body(j, c2):
            step(2 * j, rb0, sem0)
            step(2 * j + 1, rb1, sem1)
            return c2

        return lax.fori_loop(0, 8, body, c)

    lax.fori_loop(0, NB // 16, chunk, 0)
    plsc.subcore_barrier()
    pltpu.sync_copy(acc_sh.at[pl.ds(sid * RPT, RPT)],
                    out_hbm.at[pl.ds(cid * N + sid * RPT, RPT)])


@functools.partial(
    pl.kernel,
    mesh=_mesh,
    compiler_params=_sc_params,
    out_type=jax.ShapeDtypeStruct((NW, N), jnp.float32),
    scratch_types=[
        pltpu.VMEM((N,), jnp.float32),
        pltpu.VMEM((EPW,), jnp.int32),
        pltpu.VMEM((EPW,), jnp.int32),
        pltpu.VMEM((N,), jnp.float32),
    ],
)
def _sc_agg_scalar(q_hbm, src_hbm, dst_hbm, out_hbm, q_v, src_v, dst_v, acc_v):
    wid = lax.axis_index("s") * NC + lax.axis_index("c")
    pltpu.sync_copy(q_hbm, q_v)
    pltpu.sync_copy(src_hbm.at[pl.ds(wid * EPW, EPW)], src_v)
    pltpu.sync_copy(dst_hbm.at[pl.ds(wid * EPW, EPW)], dst_v)
    zero = jnp.zeros((16,), jnp.float32)

    def zbody(i, c):
        acc_v[pl.ds(i * 16, 16)] = zero
        return c

    lax.fori_loop(0, N // 16, zbody, 0)

    def body(i, c):
        s_ids = src_v[pl.ds(i * 16, 16)]
        d_ids = dst_v[pl.ds(i * 16, 16)]
        vals = plsc.load_gather(q_v, [s_ids])
        plsc.addupdate_scatter(acc_v, [d_ids], vals)
        return c

    lax.fori_loop(0, EPW // 16, body, 0)
    pltpu.sync_copy(acc_v, out_hbm.at[wid])


def _tc1_body(x_ref, w_ref, deg_ref, y_ref, dis_ref):
    deg = 1.0 + jnp.sum(deg_ref[...], axis=1, keepdims=True)
    dis = lax.rsqrt(deg)
    xw = jnp.dot(x_ref[...], w_ref[...], preferred_element_type=jnp.float32)
    y_ref[...] = xw * dis
    dis_ref[...] = dis


def _tc2_body(agg_ref, y_ref, dis_ref, b1_ref, w2_ref, q_ref):
    agg = agg_ref[0] + agg_ref[1]
    h = jnp.maximum(dis_ref[...] * (agg + y_ref[...]) + b1_ref[...], 0.0)
    q_ref[...] = jnp.dot(h, w2_ref[...],
                         preferred_element_type=jnp.float32) * dis_ref[...]


def _tc3_body(a_ref, q_ref, dis_ref, b2_ref, o_ref):
    s = jnp.sum(a_ref[...], axis=1, keepdims=True)
    o_ref[...] = dis_ref[...] * (s + q_ref[...]) + b2_ref[...]


def kernel(x, edge_index, W1, b1, W2, b2):
    ei = edge_index.astype(jnp.int32)
    src_flat = ei[0]
    dst_flat = ei[1]
    src2d = src_flat.reshape(NW * NB, B)
    dst2d = dst_flat.reshape(NW * NB, B)

    deg_parts = _sc_degree(dst_flat)

    grid = (N // RB,)
    y, dis = pl.pallas_call(
        _tc1_body,
        grid=grid,
        in_specs=[
            pl.BlockSpec((RB, F), lambda i: (i, 0)),
            pl.BlockSpec((F, F), lambda i: (0, 0)),
            pl.BlockSpec((RB, NW), lambda i: (i, 0)),
        ],
        out_specs=[
            pl.BlockSpec((RB, F), lambda i: (i, 0)),
            pl.BlockSpec((RB, 1), lambda i: (i, 0)),
        ],
        out_shape=[
            jax.ShapeDtypeStruct((N, F), jnp.float32),
            jax.ShapeDtypeStruct((N, 1), jnp.float32),
        ],
    )(x, W1, deg_parts.T)

    zeros_tile = jnp.zeros((RPT, F), jnp.float32)
    agg1 = _sc_agg_rows(y, src2d, dst2d, zeros_tile)

    q = pl.pallas_call(
        _tc2_body,
        grid=grid,
        in_specs=[
            pl.BlockSpec((NC, RB, F), lambda i: (0, i, 0)),
            pl.BlockSpec((RB, F), lambda i: (i, 0)),
            pl.BlockSpec((RB, 1), lambda i: (i, 0)),
            pl.BlockSpec((1, F), lambda i: (0, 0)),
            pl.BlockSpec((F, 1), lambda i: (0, 0)),
        ],
        out_specs=pl.BlockSpec((RB, 1), lambda i: (i, 0)),
        out_shape=jax.ShapeDtypeStruct((N, 1), jnp.float32),
    )(agg1.reshape(NC, N, F), y, dis, b1.reshape(1, F), W2)

    agg2_parts = _sc_agg_scalar(q.reshape(N), src_flat, dst_flat)

    out = pl.pallas_call(
        _tc3_body,
        grid=grid,
        in_specs=[
            pl.BlockSpec((RB, NW), lambda i: (i, 0)),
            pl.BlockSpec((RB, 1), lambda i: (i, 0)),
            pl.BlockSpec((RB, 1), lambda i: (i, 0)),
            pl.BlockSpec((1, 1), lambda i: (0, 0)),
        ],
        out_specs=pl.BlockSpec((RB, 1), lambda i: (i, 0)),
        out_shape=jax.ShapeDtypeStruct((N, 1), jnp.float32),
    )(agg2_parts.T, q, dis, b2.reshape(1, 1))

    return out.reshape(N)


# speedup vs baseline: 40.4118x; 1.1427x over previous
import functools

import jax
import jax.numpy as jnp
from jax import lax
from jax.experimental import pallas as pl
from jax.experimental.pallas import tpu as pltpu
from jax.experimental.pallas import tpu_sc as plsc

N = 10000
F = 128
E = 320000
NC = 2
NS = 16
NW = NC * NS
EPW = E // NW
B = 100
NB = EPW // B
RPT = N // NS
RB = 1000

_mesh = plsc.VectorSubcoreMesh(core_axis_name="c", subcore_axis_name="s")
_sc_params = pltpu.CompilerParams(needs_layout_passes=False,
                                  use_tc_tiling_on_sc=False)


@functools.partial(
    pl.kernel,
    mesh=_mesh,
    compiler_params=_sc_params,
    out_type=jax.ShapeDtypeStruct((NW, N), jnp.float32),
    scratch_types=[
        pltpu.VMEM((EPW,), jnp.int32),
        pltpu.VMEM((N,), jnp.float32),
    ],
)
def _sc_degree(dst_hbm, out_hbm, dst_v, acc_v):
    wid = lax.axis_index("s") * NC + lax.axis_index("c")
    pltpu.sync_copy(dst_hbm.at[pl.ds(wid * EPW, EPW)], dst_v)
    zero = jnp.zeros((16,), jnp.float32)
    one = jnp.ones((16,), jnp.float32)

    def zbody(i, c):
        acc_v[pl.ds(i * 16, 16)] = zero
        return c

    lax.fori_loop(0, N // 16, zbody, 0)

    def body(i, c):
        ids = dst_v[pl.ds(i * 16, 16)]
        plsc.addupdate_scatter(acc_v, [ids], one)
        return c

    lax.fori_loop(0, EPW // 16, body, 0)
    pltpu.sync_copy(acc_v, out_hbm.at[wid])


@functools.partial(
    pl.kernel,
    mesh=_mesh,
    compiler_params=_sc_params,
    out_type=jax.ShapeDtypeStruct((NC * N, F), jnp.float32),
    scratch_types=[
        pltpu.VMEM((NB, B), jnp.int32),
        pltpu.VMEM((NB, B), jnp.int32),
        pltpu.VMEM((B, F), jnp.float32),
        pltpu.VMEM((B, F), jnp.float32),
        pltpu.VMEM_SHARED((N, F), jnp.float32),
        pltpu.SemaphoreType.DMA,
        pltpu.SemaphoreType.DMA,
        pltpu.SemaphoreType.DMA,
        pltpu.SemaphoreType.DMA,
        pltpu.SemaphoreType.DMA,
    ],
)
def _sc_agg_rows(y_hbm, src_hbm, dst_hbm, zeros_hbm, out_hbm,
                 src_v, dst_v, rb0, rb1, acc_sh,
                 semg0, semg1, sems0, sems1, semz):
    cid = lax.axis_index("c")
    sid = lax.axis_index("s")
    wid = sid * NC + cid
    zcp = pltpu.async_copy(zeros_hbm, acc_sh.at[pl.ds(sid * RPT, RPT)], semz)
    pltpu.sync_copy(src_hbm.at[pl.ds(wid * NB, NB)], src_v)
    pltpu.sync_copy(dst_hbm.at[pl.ds(wid * NB, NB)], dst_v)
    zcp.wait()
    plsc.subcore_barrier()

    def start_gather(j, rb, sem):
        pltpu.async_copy(y_hbm.at[src_v.at[j]], rb, sem)

    def start_scatter(j, rb, sem):
        pltpu.async_copy(rb, acc_sh.at[dst_v.at[j]], sem, add=True)

    def wait_gather(rb, sem):
        pltpu.make_async_copy(y_hbm.at[src_v.at[0]], rb, sem).wait()

    def wait_scatter(rb, sem):
        pltpu.make_async_copy(rb, acc_sh.at[dst_v.at[0]], sem).wait()

    start_gather(0, rb0, semg0)
    wait_gather(rb0, semg0)
    start_scatter(0, rb0, sems0)
    start_gather(1, rb1, semg1)

    def pair(j2, c):
        j = 1 + 2 * j2
        wait_gather(rb1, semg1)
        start_scatter(j, rb1, sems1)
        wait_scatter(rb0, sems0)
        start_gather(j + 1, rb0, semg0)
        wait_gather(rb0, semg0)
        start_scatter(j + 1, rb0, sems0)
        wait_scatter(rb1, sems1)
        start_gather(j + 2, rb1, semg1)
        return c

    lax.fori_loop(0, NB // 2 - 1, pair, 0)
    wait_gather(rb1, semg1)
    start_scatter(NB - 1, rb1, sems1)
    wait_scatter(rb0, sems0)
    wait_scatter(rb1, sems1)
    plsc.subcore_barrier()
    pltpu.sync_copy(acc_sh.at[pl.ds(sid * RPT, RPT)],
                    out_hbm.at[pl.ds(cid * N + sid * RPT, RPT)])


@functools.partial(
    pl.kernel,
    mesh=_mesh,
    compiler_params=_sc_params,
    out_type=jax.ShapeDtypeStruct((NW, N), jnp.float32),
    scratch_types=[
        pltpu.VMEM((N,), jnp.float32),
        pltpu.VMEM((EPW,), jnp.int32),
        pltpu.VMEM((EPW,), jnp.int32),
        pltpu.VMEM((N,), jnp.float32),
    ],
)
def _sc_agg_scalar(q_hbm, src_hbm, dst_hbm, out_hbm, q_v, src_v, dst_v, acc_v):
    wid = lax.axis_index("s") * NC + lax.axis_index("c")
    pltpu.sync_copy(q_hbm, q_v)
    pltpu.sync_copy(src_hbm.at[pl.ds(wid * EPW, EPW)], src_v)
    pltpu.sync_copy(dst_hbm.at[pl.ds(wid * EPW, EPW)], dst_v)
    zero = jnp.zeros((16,), jnp.float32)

    def zbody(i, c):
        acc_v[pl.ds(i * 16, 16)] = zero
        return c

    lax.fori_loop(0, N // 16, zbody, 0)

    def body(i, c):
        s_ids = src_v[pl.ds(i * 16, 16)]
        d_ids = dst_v[pl.ds(i * 16, 16)]
        vals = plsc.load_gather(q_v, [s_ids])
        plsc.addupdate_scatter(acc_v, [d_ids], vals)
        return c

    lax.fori_loop(0, EPW // 16, body, 0)
    pltpu.sync_copy(acc_v, out_hbm.at[wid])


def _tc1_body(x_ref, w_ref, deg_ref, y_ref, dis_ref):
    deg = 1.0 + jnp.sum(deg_ref[...], axis=1, keepdims=True)
    dis = lax.rsqrt(deg)
    xw = jnp.dot(x_ref[...], w_ref[...], preferred_element_type=jnp.float32)
    y_ref[...] = xw * dis
    dis_ref[...] = dis


def _tc2_body(agg_ref, y_ref, dis_ref, b1_ref, w2_ref, q_ref):
    agg = agg_ref[0] + agg_ref[1]
    h = jnp.maximum(dis_ref[...] * (agg + y_ref[...]) + b1_ref[...], 0.0)
    q_ref[...] = jnp.dot(h, w2_ref[...],
                         preferred_element_type=jnp.float32) * dis_ref[...]


def _tc3_body(a_ref, q_ref, dis_ref, b2_ref, o_ref):
    s = jnp.sum(a_ref[...], axis=1, keepdims=True)
    o_ref[...] = dis_ref[...] * (s + q_ref[...]) + b2_ref[...]


def kernel(x, edge_index, W1, b1, W2, b2):
    ei = edge_index.astype(jnp.int32)
    src_flat = ei[0]
    dst_flat = ei[1]
    src2d = src_flat.reshape(NW * NB, B)
    dst2d = dst_flat.reshape(NW * NB, B)

    deg_parts = _sc_degree(dst_flat)

    grid = (N // RB,)
    y, dis = pl.pallas_call(
        _tc1_body,
        grid=grid,
        in_specs=[
            pl.BlockSpec((RB, F), lambda i: (i, 0)),
            pl.BlockSpec((F, F), lambda i: (0, 0)),
            pl.BlockSpec((RB, NW), lambda i: (i, 0)),
        ],
        out_specs=[
            pl.BlockSpec((RB, F), lambda i: (i, 0)),
            pl.BlockSpec((RB, 1), lambda i: (i, 0)),
        ],
        out_shape=[
            jax.ShapeDtypeStruct((N, F), jnp.float32),
            jax.ShapeDtypeStruct((N, 1), jnp.float32),
        ],
    )(x, W1, deg_parts.T)

    zeros_tile = jnp.zeros((RPT, F), jnp.float32)
    agg1 = _sc_agg_rows(y, src2d, dst2d, zeros_tile)

    q = pl.pallas_call(
        _tc2_body,
        grid=grid,
        in_specs=[
            pl.BlockSpec((NC, RB, F), lambda i: (0, i, 0)),
            pl.BlockSpec((RB, F), lambda i: (i, 0)),
            pl.BlockSpec((RB, 1), lambda i: (i, 0)),
            pl.BlockSpec((1, F), lambda i: (0, 0)),
            pl.BlockSpec((F, 1), lambda i: (0, 0)),
        ],
        out_specs=pl.BlockSpec((RB, 1), lambda i: (i, 0)),
        out_shape=jax.ShapeDtypeStruct((N, 1), jnp.float32),
    )(agg1.reshape(NC, N, F), y, dis, b1.reshape(1, F), W2)

    agg2_parts = _sc_agg_scalar(q.reshape(N), src_flat, dst_flat)

    out = pl.pallas_call(
        _tc3_body,
        grid=grid,
        in_specs=[
            pl.BlockSpec((RB, NW), lambda i: (i, 0)),
            pl.BlockSpec((RB, 1), lambda i: (i, 0)),
            pl.BlockSpec((RB, 1), lambda i: (i, 0)),
            pl.BlockSpec((1, 1), lambda i: (0, 0)),
        ],
        out_specs=pl.BlockSpec((RB, 1), lambda i: (i, 0)),
        out_shape=jax.ShapeDtypeStruct((N, 1), jnp.float32),
    )(agg2_parts.T, q, dis, b2.reshape(1, 1))

    return out.reshape(N)
